# baseline (speedup 1.0000x reference)
import functools

import jax
import jax.numpy as jnp
from jax import lax
from jax.experimental import pallas as pl
from jax.experimental.pallas import tpu as pltpu
from jax.experimental.pallas import tpu_sc as plsc

HIDDEN = 128
MAX_SENT = 100
BATCH = 4096
NUM_ELEM = 100
EPS = 1e-5

BB = 128
CH = 8


def _sc_gather(ids_hbm, table_hbm, out_hbm, idx_v, rows_v, sem):
    wid = lax.axis_index("s") * 2 + lax.axis_index("c")

    @pl.when(wid == 0)
    def _():
        pltpu.sync_copy(ids_hbm, idx_v)
        pltpu.async_copy(table_hbm.at[idx_v], rows_v, sem).wait()
        pltpu.sync_copy(rows_v, out_hbm)


_sc_gather_call = functools.partial(
    pl.kernel,
    out_type=jax.ShapeDtypeStruct((NUM_ELEM, HIDDEN), jnp.float32),
    mesh=plsc.VectorSubcoreMesh(core_axis_name="c", subcore_axis_name="s"),
    scratch_types=[
        pltpu.VMEM((NUM_ELEM,), jnp.int32),
        pltpu.VMEM((NUM_ELEM, HIDDEN), jnp.float32),
        pltpu.SemaphoreType.DMA,
    ],
)(_sc_gather)


def _tc_body(x_ref, pos_ref, gamma_ref, beta_ref, o_ref):
    pos = pos_ref[...]
    gamma = gamma_ref[0, :]
    beta = beta_ref[0, :]

    def step(k, _):
        x = x_ref[pl.ds(k * CH, CH), :, :]
        out = x + pos[None, :, :]
        mean = jnp.mean(out, axis=-1, keepdims=True)
        c = out - mean
        var = jnp.mean(c * c, axis=-1, keepdims=True)
        normed = c * lax.rsqrt(var + EPS)
        o_ref[pl.ds(k * CH, CH), :, :] = normed * gamma + beta
        return 0

    lax.fori_loop(0, BB // CH, step, 0)


@jax.jit
def kernel(batch_elem_emb, sent_pos_ids, emb_table, gamma, beta):
    ids = sent_pos_ids.astype(jnp.int32)
    pos = _sc_gather_call(ids, emb_table)
    gamma2 = gamma.reshape(1, HIDDEN)
    beta2 = beta.reshape(1, HIDDEN)
    return pl.pallas_call(
        _tc_body,
        grid=(BATCH // BB,),
        in_specs=[
            pl.BlockSpec((BB, NUM_ELEM, HIDDEN), lambda i: (i, 0, 0)),
            pl.BlockSpec((NUM_ELEM, HIDDEN), lambda i: (0, 0)),
            pl.BlockSpec((1, HIDDEN), lambda i: (0, 0)),
            pl.BlockSpec((1, HIDDEN), lambda i: (0, 0)),
        ],
        out_specs=pl.BlockSpec((BB, NUM_ELEM, HIDDEN), lambda i: (i, 0, 0)),
        out_shape=jax.ShapeDtypeStruct((BATCH, NUM_ELEM, HIDDEN), jnp.float32),
    )(batch_elem_emb, pos, gamma2, beta2)


# speedup vs baseline: 1.0024x; 1.0024x over previous
import functools

import jax
import jax.numpy as jnp
from jax import lax
from jax.experimental import pallas as pl
from jax.experimental.pallas import tpu as pltpu
from jax.experimental.pallas import tpu_sc as plsc

HIDDEN = 128
MAX_SENT = 100
BATCH = 4096
NUM_ELEM = 100
EPS = 1e-5

BB = 128
CH = 8


def _sc_gather(ids_hbm, table_hbm, out_hbm, idx_v, rows_v, sem):
    wid = lax.axis_index("s") * 2 + lax.axis_index("c")

    @pl.when(wid == 0)
    def _():
        pltpu.sync_copy(ids_hbm, idx_v)
        pltpu.async_copy(table_hbm.at[idx_v], rows_v, sem).wait()
        pltpu.sync_copy(rows_v, out_hbm)


_sc_gather_call = functools.partial(
    pl.kernel,
    out_type=jax.ShapeDtypeStruct((NUM_ELEM, HIDDEN), jnp.float32),
    mesh=plsc.VectorSubcoreMesh(core_axis_name="c", subcore_axis_name="s"),
    scratch_types=[
        pltpu.VMEM((NUM_ELEM,), jnp.int32),
        pltpu.VMEM((NUM_ELEM, HIDDEN), jnp.float32),
        pltpu.SemaphoreType.DMA,
    ],
)(_sc_gather)


def _tc_body(x_ref, pos_ref, gamma_ref, beta_ref, o_ref):
    pos = pos_ref[...]
    gamma = gamma_ref[0, :]
    beta = beta_ref[0, :]

    def step(k, _):
        x = x_ref[pl.ds(k * CH, CH), :, :]
        out = x + pos[None, :, :]
        mean = jnp.mean(out, axis=-1, keepdims=True)
        c = out - mean
        var = jnp.mean(c * c, axis=-1, keepdims=True)
        normed = c * lax.rsqrt(var + EPS)
        o_ref[pl.ds(k * CH, CH), :, :] = normed * gamma + beta
        return 0

    lax.fori_loop(0, BB // CH, step, 0)


@jax.jit
def kernel(batch_elem_emb, sent_pos_ids, emb_table, gamma, beta):
    ids = sent_pos_ids.astype(jnp.int32)
    pos = _sc_gather_call(ids, emb_table)
    gamma2 = gamma.reshape(1, HIDDEN)
    beta2 = beta.reshape(1, HIDDEN)
    return pl.pallas_call(
        _tc_body,
        grid=(BATCH // BB,),
        in_specs=[
            pl.BlockSpec((BB, NUM_ELEM, HIDDEN), lambda i: (i, 0, 0)),
            pl.BlockSpec((NUM_ELEM, HIDDEN), lambda i: (0, 0)),
            pl.BlockSpec((1, HIDDEN), lambda i: (0, 0)),
            pl.BlockSpec((1, HIDDEN), lambda i: (0, 0)),
        ],
        out_specs=pl.BlockSpec((BB, NUM_ELEM, HIDDEN), lambda i: (i, 0, 0)),
        out_shape=jax.ShapeDtypeStruct((BATCH, NUM_ELEM, HIDDEN), jnp.float32),
        compiler_params=pltpu.CompilerParams(
            dimension_semantics=("parallel",),
        ),
    )(batch_elem_emb, pos, gamma2, beta2)
